# baseline (speedup 1.0000x reference)
import jax
import jax.numpy as jnp
from jax.experimental import pallas as pl
from jax.experimental.pallas import tpu as pltpu

_K = 176
_STYLE_OFF = 112
_TPD_SUB = 52
_ONE_SUB = 55


def _prep_kernel(inst_pad_ref, wpi_t_ref, style_pad_ref, wps_t_ref,
                 v2_ref, wptpd_t_ref, e_one_ref, b_proj_ref, a_ref):
    a_ref[0:_STYLE_OFF, :] = jnp.dot(
        inst_pad_ref[...], wpi_t_ref[...],
        preferred_element_type=jnp.float32).astype(jnp.bfloat16)
    block2 = (jnp.dot(style_pad_ref[...], wps_t_ref[...],
                      preferred_element_type=jnp.float32)
              + jnp.dot(v2_ref[...], wptpd_t_ref[...],
                        preferred_element_type=jnp.float32)
              + jnp.dot(e_one_ref[...], b_proj_ref[...],
                        preferred_element_type=jnp.float32))
    a_ref[_STYLE_OFF:_K, :] = block2.astype(jnp.bfloat16)


def _main_kernel(inst_ref, scal_ref,
                 inst_pad_ref, wpi_t_ref, style_pad_ref, wps_t_ref,
                 v2_ref, wptpd_t_ref, e_one_ref, b_proj_ref,
                 out_ref, a_ref):
    @pl.when(pl.program_id(0) == 0)
    def _prep():
        a_ref[0:_STYLE_OFF, :] = jnp.dot(
            inst_pad_ref[...], wpi_t_ref[...],
            preferred_element_type=jnp.float32).astype(jnp.bfloat16)
        block2 = (jnp.dot(style_pad_ref[...], wps_t_ref[...],
                          preferred_element_type=jnp.float32)
                  + jnp.dot(v2_ref[...], wptpd_t_ref[...],
                            preferred_element_type=jnp.float32)
                  + jnp.dot(e_one_ref[...], b_proj_ref[...],
                            preferred_element_type=jnp.float32))
        a_ref[_STYLE_OFF:_K, :] = block2.astype(jnp.bfloat16)

    idx = inst_ref[...].astype(jnp.bfloat16)
    L, bsz = idx.shape
    iota_i = jax.lax.broadcasted_iota(jnp.int32, (_STYLE_OFF, bsz), 0).astype(
        jnp.bfloat16)
    mi = jnp.zeros((_STYLE_OFF, bsz), jnp.bfloat16)
    one = jnp.ones((), jnp.bfloat16)
    for l in range(L):
        mi = jnp.where(idx[l:l + 1, :] == iota_i, mi + one, mi)
    iota_s = jax.lax.broadcasted_iota(jnp.int32, (_K - _STYLE_OFF, bsz),
                                      0).astype(jnp.bfloat16)
    scal = scal_ref[...].astype(jnp.bfloat16)
    m2 = (scal[0:1, :] == iota_s).astype(jnp.bfloat16)
    m2 = jnp.where(iota_s == _TPD_SUB, scal[1:2, :], m2)
    m2 = jnp.where(iota_s == _TPD_SUB + 1, scal[2:3, :], m2)
    m2 = jnp.where(iota_s == _TPD_SUB + 2, scal[3:4, :], m2)
    m2 = jnp.where(iota_s == _ONE_SUB, one, m2)
    m = jnp.concatenate([mi, m2], axis=0)
    out_ref[...] = jax.lax.dot_general(
        m, a_ref[...],
        dimension_numbers=(((0,), (0,)), ((), ())),
        preferred_element_type=jnp.float32)


def kernel(instruments, style, tempo, pitch, duration,
           instrument_table, style_table,
           W_tempo, b_tempo, W_pitch, b_pitch, W_dur, b_dur,
           W_proj, b_proj):
    B, L = instruments.shape
    n_inst, d_inst = instrument_table.shape
    n_style, d_style = style_table.shape
    H = W_proj.shape[0]

    f32 = jnp.float32
    inst_pad = jnp.zeros((_STYLE_OFF, d_inst), f32).at[:n_inst].set(
        instrument_table)
    style_pad = jnp.zeros((64, d_style), f32).at[:n_style].set(style_table)
    wpi_t = W_proj[:, :d_inst].T
    wps_t = W_proj[:, d_inst:d_inst + d_style].T
    wptpd_t = W_proj[:, d_inst + d_style:].T
    v2 = jnp.zeros((64, 192), f32)
    v2 = v2.at[_TPD_SUB, 0:64].set(W_tempo[:, 0])
    v2 = v2.at[_TPD_SUB + 1, 64:128].set(W_pitch[:, 0])
    v2 = v2.at[_TPD_SUB + 2, 128:192].set(W_dur[:, 0])
    v2 = v2.at[_ONE_SUB, 0:64].add(b_tempo)
    v2 = v2.at[_ONE_SUB, 64:128].add(b_pitch)
    v2 = v2.at[_ONE_SUB, 128:192].add(b_dur)
    e_one = jnp.zeros((64, 8), f32).at[_ONE_SUB, 0].set(1.0)
    b_proj2 = jnp.zeros((8, H), f32).at[0].set(b_proj)

    inst_t = instruments.astype(jnp.int32).T
    scal = jnp.stack([style.astype(f32), tempo[:, 0], pitch[:, 0],
                      duration[:, 0]], axis=0)
    bsz = 2048
    grid = (B // bsz,)
    out = pl.pallas_call(
        _main_kernel,
        grid=grid,
        in_specs=[
            pl.BlockSpec((L, bsz), lambda i: (0, i)),
            pl.BlockSpec((4, bsz), lambda i: (0, i)),
            pl.BlockSpec((_STYLE_OFF, d_inst), lambda i: (0, 0)),
            pl.BlockSpec((d_inst, H), lambda i: (0, 0)),
            pl.BlockSpec((64, d_style), lambda i: (0, 0)),
            pl.BlockSpec((d_style, H), lambda i: (0, 0)),
            pl.BlockSpec((64, 192), lambda i: (0, 0)),
            pl.BlockSpec((192, H), lambda i: (0, 0)),
            pl.BlockSpec((64, 8), lambda i: (0, 0)),
            pl.BlockSpec((8, H), lambda i: (0, 0)),
        ],
        out_specs=pl.BlockSpec((bsz, H), lambda i: (i, 0)),
        out_shape=jax.ShapeDtypeStruct((B, H), f32),
        scratch_shapes=[pltpu.VMEM((_K, H), jnp.bfloat16)],
        compiler_params=pltpu.CompilerParams(
            dimension_semantics=("arbitrary",)),
    )(inst_t, scal,
      inst_pad, wpi_t, style_pad, wps_t, v2, wptpd_t, e_one, b_proj2)
    return out


# speedup vs baseline: 1.0810x; 1.0810x over previous
import jax
import jax.numpy as jnp
from jax.experimental import pallas as pl
from jax.experimental.pallas import tpu as pltpu

_K = 176
_STYLE_OFF = 112
_TPD_SUB = 52
_ONE_SUB = 55


def _prep_kernel(inst_pad_ref, wpi_t_ref, style_pad_ref, wps_t_ref,
                 v2_ref, wptpd_t_ref, e_one_ref, b_proj_ref, a_ref):
    a_ref[0:_STYLE_OFF, :] = jnp.dot(
        inst_pad_ref[...], wpi_t_ref[...],
        preferred_element_type=jnp.float32).astype(jnp.bfloat16)
    block2 = (jnp.dot(style_pad_ref[...], wps_t_ref[...],
                      preferred_element_type=jnp.float32)
              + jnp.dot(v2_ref[...], wptpd_t_ref[...],
                        preferred_element_type=jnp.float32)
              + jnp.dot(e_one_ref[...], b_proj_ref[...],
                        preferred_element_type=jnp.float32))
    a_ref[_STYLE_OFF:_K, :] = block2.astype(jnp.bfloat16)


def _main_kernel(inst_ref, sty_ref, tempo_ref, pitch_ref, dur_ref,
                 inst_pad_ref, wpi_t_ref, style_pad_ref, wps_t_ref,
                 v2_ref, wptpd_t_ref, e_one_ref, b_proj_ref,
                 out_ref, a_ref):
    @pl.when(pl.program_id(0) == 0)
    def _prep():
        a_ref[0:_STYLE_OFF, :] = jnp.dot(
            inst_pad_ref[...], wpi_t_ref[...],
            preferred_element_type=jnp.float32).astype(jnp.bfloat16)
        block2 = (jnp.dot(style_pad_ref[...], wps_t_ref[...],
                          preferred_element_type=jnp.float32)
                  + jnp.dot(v2_ref[...], wptpd_t_ref[...],
                            preferred_element_type=jnp.float32)
                  + jnp.dot(e_one_ref[...], b_proj_ref[...],
                            preferred_element_type=jnp.float32))
        a_ref[_STYLE_OFF:_K, :] = block2.astype(jnp.bfloat16)

    idx = inst_ref[...].astype(jnp.bfloat16)
    L, bsz = idx.shape
    iota_i = jax.lax.broadcasted_iota(jnp.int32, (_STYLE_OFF, bsz), 0).astype(
        jnp.bfloat16)
    mi = jnp.zeros((_STYLE_OFF, bsz), jnp.bfloat16)
    one = jnp.ones((), jnp.bfloat16)
    for l in range(L):
        mi = jnp.where(idx[l:l + 1, :] == iota_i, mi + one, mi)
    iota_s = jax.lax.broadcasted_iota(jnp.int32, (_K - _STYLE_OFF, bsz),
                                      0).astype(jnp.bfloat16)
    m2 = (sty_ref[...].astype(jnp.bfloat16) == iota_s).astype(jnp.bfloat16)
    m2 = jnp.where(iota_s == _TPD_SUB, tempo_ref[...].astype(jnp.bfloat16), m2)
    m2 = jnp.where(iota_s == _TPD_SUB + 1,
                   pitch_ref[...].astype(jnp.bfloat16), m2)
    m2 = jnp.where(iota_s == _TPD_SUB + 2,
                   dur_ref[...].astype(jnp.bfloat16), m2)
    m2 = jnp.where(iota_s == _ONE_SUB, one, m2)
    m = jnp.concatenate([mi, m2], axis=0)
    out_ref[...] = jax.lax.dot_general(
        m, a_ref[...],
        dimension_numbers=(((0,), (0,)), ((), ())),
        preferred_element_type=jnp.float32)


def kernel(instruments, style, tempo, pitch, duration,
           instrument_table, style_table,
           W_tempo, b_tempo, W_pitch, b_pitch, W_dur, b_dur,
           W_proj, b_proj):
    B, L = instruments.shape
    n_inst, d_inst = instrument_table.shape
    n_style, d_style = style_table.shape
    H = W_proj.shape[0]

    f32 = jnp.float32
    inst_pad = jnp.zeros((_STYLE_OFF, d_inst), f32).at[:n_inst].set(
        instrument_table)
    style_pad = jnp.zeros((64, d_style), f32).at[:n_style].set(style_table)
    wpi_t = W_proj[:, :d_inst].T
    wps_t = W_proj[:, d_inst:d_inst + d_style].T
    wptpd_t = W_proj[:, d_inst + d_style:].T
    v2 = jnp.zeros((64, 192), f32)
    v2 = v2.at[_TPD_SUB, 0:64].set(W_tempo[:, 0])
    v2 = v2.at[_TPD_SUB + 1, 64:128].set(W_pitch[:, 0])
    v2 = v2.at[_TPD_SUB + 2, 128:192].set(W_dur[:, 0])
    v2 = v2.at[_ONE_SUB, 0:64].add(b_tempo)
    v2 = v2.at[_ONE_SUB, 64:128].add(b_pitch)
    v2 = v2.at[_ONE_SUB, 128:192].add(b_dur)
    e_one = jnp.zeros((64, 8), f32).at[_ONE_SUB, 0].set(1.0)
    b_proj2 = jnp.zeros((8, H), f32).at[0].set(b_proj)

    inst_t = instruments.astype(jnp.int32).T
    sty_t = style.reshape(1, B).astype(jnp.int32)
    tempo_t = tempo.reshape(1, B)
    pitch_t = pitch.reshape(1, B)
    dur_t = duration.reshape(1, B)
    bsz = 2048
    grid = (B // bsz,)
    out = pl.pallas_call(
        _main_kernel,
        grid=grid,
        in_specs=[
            pl.BlockSpec((L, bsz), lambda i: (0, i)),
            pl.BlockSpec((1, bsz), lambda i: (0, i)),
            pl.BlockSpec((1, bsz), lambda i: (0, i)),
            pl.BlockSpec((1, bsz), lambda i: (0, i)),
            pl.BlockSpec((1, bsz), lambda i: (0, i)),
            pl.BlockSpec((_STYLE_OFF, d_inst), lambda i: (0, 0)),
            pl.BlockSpec((d_inst, H), lambda i: (0, 0)),
            pl.BlockSpec((64, d_style), lambda i: (0, 0)),
            pl.BlockSpec((d_style, H), lambda i: (0, 0)),
            pl.BlockSpec((64, 192), lambda i: (0, 0)),
            pl.BlockSpec((192, H), lambda i: (0, 0)),
            pl.BlockSpec((64, 8), lambda i: (0, 0)),
            pl.BlockSpec((8, H), lambda i: (0, 0)),
        ],
        out_specs=pl.BlockSpec((bsz, H), lambda i: (i, 0)),
        out_shape=jax.ShapeDtypeStruct((B, H), f32),
        scratch_shapes=[pltpu.VMEM((_K, H), jnp.bfloat16)],
        compiler_params=pltpu.CompilerParams(
            dimension_semantics=("arbitrary",)),
    )(inst_t, sty_t, tempo_t, pitch_t, dur_t,
      inst_pad, wpi_t, style_pad, wps_t, v2, wptpd_t, e_one, b_proj2)
    return out


# speedup vs baseline: 1.1107x; 1.0275x over previous
import jax
import jax.numpy as jnp
from jax.experimental import pallas as pl
from jax.experimental.pallas import tpu as pltpu

_K = 176
_STYLE_OFF = 112
_TPD_SUB = 52
_ONE_SUB = 55


def _prep_kernel(inst_pad_ref, wpi_t_ref, style_pad_ref, wps_t_ref,
                 v2_ref, wptpd_t_ref, e_one_ref, b_proj_ref, a_ref):
    a_ref[0:_STYLE_OFF, :] = jnp.dot(
        inst_pad_ref[...], wpi_t_ref[...],
        preferred_element_type=jnp.float32).astype(jnp.bfloat16)
    block2 = (jnp.dot(style_pad_ref[...], wps_t_ref[...],
                      preferred_element_type=jnp.float32)
              + jnp.dot(v2_ref[...], wptpd_t_ref[...],
                        preferred_element_type=jnp.float32)
              + jnp.dot(e_one_ref[...], b_proj_ref[...],
                        preferred_element_type=jnp.float32))
    a_ref[_STYLE_OFF:_K, :] = block2.astype(jnp.bfloat16)


def _main_kernel(inst_ref, sty_ref, tempo_ref, pitch_ref, dur_ref,
                 inst_pad_ref, wpi_t_ref, style_pad_ref, wps_t_ref,
                 v2_ref, wptpd_t_ref, e_one_ref, b_proj_ref,
                 out_ref, a_ref):
    @pl.when(pl.program_id(0) == 0)
    def _prep():
        a_ref[0:_STYLE_OFF, :] = jnp.dot(
            inst_pad_ref[...], wpi_t_ref[...],
            preferred_element_type=jnp.float32).astype(jnp.bfloat16)
        block2 = (jnp.dot(style_pad_ref[...], wps_t_ref[...],
                          preferred_element_type=jnp.float32)
                  + jnp.dot(v2_ref[...], wptpd_t_ref[...],
                            preferred_element_type=jnp.float32)
                  + jnp.dot(e_one_ref[...], b_proj_ref[...],
                            preferred_element_type=jnp.float32))
        a_ref[_STYLE_OFF:_K, :] = block2.astype(jnp.bfloat16)

    idx_all = inst_ref[...].astype(jnp.bfloat16)
    L, bsz = idx_all.shape
    hsz = bsz // 2
    one = jnp.ones((), jnp.bfloat16)
    for h in range(2):
        lo, hi = h * hsz, (h + 1) * hsz
        idx = idx_all[:, lo:hi]
        iota_i = jax.lax.broadcasted_iota(
            jnp.int32, (_STYLE_OFF, hsz), 0).astype(jnp.bfloat16)
        mi = jnp.zeros((_STYLE_OFF, hsz), jnp.bfloat16)
        for l in range(L):
            mi = jnp.where(idx[l:l + 1, :] == iota_i, mi + one, mi)
        iota_s = jax.lax.broadcasted_iota(
            jnp.int32, (_K - _STYLE_OFF, hsz), 0).astype(jnp.bfloat16)
        m2 = (sty_ref[:, lo:hi].astype(jnp.bfloat16) == iota_s).astype(
            jnp.bfloat16)
        m2 = jnp.where(iota_s == _TPD_SUB,
                       tempo_ref[:, lo:hi].astype(jnp.bfloat16), m2)
        m2 = jnp.where(iota_s == _TPD_SUB + 1,
                       pitch_ref[:, lo:hi].astype(jnp.bfloat16), m2)
        m2 = jnp.where(iota_s == _TPD_SUB + 2,
                       dur_ref[:, lo:hi].astype(jnp.bfloat16), m2)
        m2 = jnp.where(iota_s == _ONE_SUB, one, m2)
        m = jnp.concatenate([mi, m2], axis=0)
        out_ref[lo:hi, :] = jax.lax.dot_general(
            m, a_ref[...],
            dimension_numbers=(((0,), (0,)), ((), ())),
            preferred_element_type=jnp.float32)


def kernel(instruments, style, tempo, pitch, duration,
           instrument_table, style_table,
           W_tempo, b_tempo, W_pitch, b_pitch, W_dur, b_dur,
           W_proj, b_proj):
    B, L = instruments.shape
    n_inst, d_inst = instrument_table.shape
    n_style, d_style = style_table.shape
    H = W_proj.shape[0]

    f32 = jnp.float32
    inst_pad = jnp.zeros((_STYLE_OFF, d_inst), f32).at[:n_inst].set(
        instrument_table)
    style_pad = jnp.zeros((64, d_style), f32).at[:n_style].set(style_table)
    wpi_t = W_proj[:, :d_inst].T
    wps_t = W_proj[:, d_inst:d_inst + d_style].T
    wptpd_t = W_proj[:, d_inst + d_style:].T
    v2 = jnp.zeros((64, 192), f32)
    v2 = v2.at[_TPD_SUB, 0:64].set(W_tempo[:, 0])
    v2 = v2.at[_TPD_SUB + 1, 64:128].set(W_pitch[:, 0])
    v2 = v2.at[_TPD_SUB + 2, 128:192].set(W_dur[:, 0])
    v2 = v2.at[_ONE_SUB, 0:64].add(b_tempo)
    v2 = v2.at[_ONE_SUB, 64:128].add(b_pitch)
    v2 = v2.at[_ONE_SUB, 128:192].add(b_dur)
    e_one = jnp.zeros((64, 8), f32).at[_ONE_SUB, 0].set(1.0)
    b_proj2 = jnp.zeros((8, H), f32).at[0].set(b_proj)

    inst_t = instruments.astype(jnp.int32).T
    sty_t = style.reshape(1, B).astype(jnp.int32)
    tempo_t = tempo.reshape(1, B)
    pitch_t = pitch.reshape(1, B)
    dur_t = duration.reshape(1, B)
    bsz = 2048
    grid = (B // bsz,)
    out = pl.pallas_call(
        _main_kernel,
        grid=grid,
        in_specs=[
            pl.BlockSpec((L, bsz), lambda i: (0, i)),
            pl.BlockSpec((1, bsz), lambda i: (0, i)),
            pl.BlockSpec((1, bsz), lambda i: (0, i)),
            pl.BlockSpec((1, bsz), lambda i: (0, i)),
            pl.BlockSpec((1, bsz), lambda i: (0, i)),
            pl.BlockSpec((_STYLE_OFF, d_inst), lambda i: (0, 0)),
            pl.BlockSpec((d_inst, H), lambda i: (0, 0)),
            pl.BlockSpec((64, d_style), lambda i: (0, 0)),
            pl.BlockSpec((d_style, H), lambda i: (0, 0)),
            pl.BlockSpec((64, 192), lambda i: (0, 0)),
            pl.BlockSpec((192, H), lambda i: (0, 0)),
            pl.BlockSpec((64, 8), lambda i: (0, 0)),
            pl.BlockSpec((8, H), lambda i: (0, 0)),
        ],
        out_specs=pl.BlockSpec((bsz, H), lambda i: (i, 0)),
        out_shape=jax.ShapeDtypeStruct((B, H), f32),
        scratch_shapes=[pltpu.VMEM((_K, H), jnp.bfloat16)],
        compiler_params=pltpu.CompilerParams(
            dimension_semantics=("arbitrary",)),
    )(inst_t, sty_t, tempo_t, pitch_t, dur_t,
      inst_pad, wpi_t, style_pad, wps_t, v2, wptpd_t, e_one, b_proj2)
    return out


# speedup vs baseline: 1.1242x; 1.0121x over previous
import jax
import jax.numpy as jnp
from jax.experimental import pallas as pl
from jax.experimental.pallas import tpu as pltpu

_K = 176
_STYLE_OFF = 112
_TPD_SUB = 52
_ONE_SUB = 55


def _prep_kernel(inst_pad_ref, wpi_t_ref, style_pad_ref, wps_t_ref,
                 v2_ref, wptpd_t_ref, e_one_ref, b_proj_ref, a_ref):
    a_ref[0:_STYLE_OFF, :] = jnp.dot(
        inst_pad_ref[...], wpi_t_ref[...],
        preferred_element_type=jnp.float32).astype(jnp.bfloat16)
    block2 = (jnp.dot(style_pad_ref[...], wps_t_ref[...],
                      preferred_element_type=jnp.float32)
              + jnp.dot(v2_ref[...], wptpd_t_ref[...],
                        preferred_element_type=jnp.float32)
              + jnp.dot(e_one_ref[...], b_proj_ref[...],
                        preferred_element_type=jnp.float32))
    a_ref[_STYLE_OFF:_K, :] = block2.astype(jnp.bfloat16)


def _main_kernel(inst_ref, sty_ref, tempo_ref, pitch_ref, dur_ref,
                 inst_pad_ref, wpi_t_ref, style_pad_ref, wps_t_ref,
                 v2_ref, wptpd_t_ref, e_one_ref, b_proj_ref,
                 out_ref, a_ref):
    @pl.when(pl.program_id(0) == 0)
    def _prep():
        a_ref[0:_STYLE_OFF, :] = jnp.dot(
            inst_pad_ref[...], wpi_t_ref[...],
            preferred_element_type=jnp.float32).astype(jnp.bfloat16)
        block2 = (jnp.dot(style_pad_ref[...], wps_t_ref[...],
                          preferred_element_type=jnp.float32)
                  + jnp.dot(v2_ref[...], wptpd_t_ref[...],
                            preferred_element_type=jnp.float32)
                  + jnp.dot(e_one_ref[...], b_proj_ref[...],
                            preferred_element_type=jnp.float32))
        a_ref[_STYLE_OFF:_K, :] = block2.astype(jnp.bfloat16)

    idx_all = inst_ref[...].astype(jnp.bfloat16)
    L, bsz = idx_all.shape
    hsz = bsz // 4
    one = jnp.ones((), jnp.bfloat16)
    for h in range(4):
        lo, hi = h * hsz, (h + 1) * hsz
        idx = idx_all[:, lo:hi]
        iota_i = jax.lax.broadcasted_iota(
            jnp.int32, (_STYLE_OFF, hsz), 0).astype(jnp.bfloat16)
        mi = jnp.zeros((_STYLE_OFF, hsz), jnp.bfloat16)
        for l in range(L):
            mi = jnp.where(idx[l:l + 1, :] == iota_i, mi + one, mi)
        iota_s = jax.lax.broadcasted_iota(
            jnp.int32, (_K - _STYLE_OFF, hsz), 0).astype(jnp.bfloat16)
        m2 = (sty_ref[:, lo:hi].astype(jnp.bfloat16) == iota_s).astype(
            jnp.bfloat16)
        m2 = jnp.where(iota_s == _TPD_SUB,
                       tempo_ref[:, lo:hi].astype(jnp.bfloat16), m2)
        m2 = jnp.where(iota_s == _TPD_SUB + 1,
                       pitch_ref[:, lo:hi].astype(jnp.bfloat16), m2)
        m2 = jnp.where(iota_s == _TPD_SUB + 2,
                       dur_ref[:, lo:hi].astype(jnp.bfloat16), m2)
        m2 = jnp.where(iota_s == _ONE_SUB, one, m2)
        m = jnp.concatenate([mi, m2], axis=0)
        out_ref[lo:hi, :] = jax.lax.dot_general(
            m, a_ref[...],
            dimension_numbers=(((0,), (0,)), ((), ())),
            preferred_element_type=jnp.float32)


def kernel(instruments, style, tempo, pitch, duration,
           instrument_table, style_table,
           W_tempo, b_tempo, W_pitch, b_pitch, W_dur, b_dur,
           W_proj, b_proj):
    B, L = instruments.shape
    n_inst, d_inst = instrument_table.shape
    n_style, d_style = style_table.shape
    H = W_proj.shape[0]

    f32 = jnp.float32
    inst_pad = jnp.zeros((_STYLE_OFF, d_inst), f32).at[:n_inst].set(
        instrument_table)
    style_pad = jnp.zeros((64, d_style), f32).at[:n_style].set(style_table)
    wpi_t = W_proj[:, :d_inst].T
    wps_t = W_proj[:, d_inst:d_inst + d_style].T
    wptpd_t = W_proj[:, d_inst + d_style:].T
    v2 = jnp.zeros((64, 192), f32)
    v2 = v2.at[_TPD_SUB, 0:64].set(W_tempo[:, 0])
    v2 = v2.at[_TPD_SUB + 1, 64:128].set(W_pitch[:, 0])
    v2 = v2.at[_TPD_SUB + 2, 128:192].set(W_dur[:, 0])
    v2 = v2.at[_ONE_SUB, 0:64].add(b_tempo)
    v2 = v2.at[_ONE_SUB, 64:128].add(b_pitch)
    v2 = v2.at[_ONE_SUB, 128:192].add(b_dur)
    e_one = jnp.zeros((64, 8), f32).at[_ONE_SUB, 0].set(1.0)
    b_proj2 = jnp.zeros((8, H), f32).at[0].set(b_proj)

    inst_t = instruments.astype(jnp.int32).T
    sty_t = style.reshape(1, B).astype(jnp.int32)
    tempo_t = tempo.reshape(1, B)
    pitch_t = pitch.reshape(1, B)
    dur_t = duration.reshape(1, B)
    bsz = 2048
    grid = (B // bsz,)
    out = pl.pallas_call(
        _main_kernel,
        grid=grid,
        in_specs=[
            pl.BlockSpec((L, bsz), lambda i: (0, i)),
            pl.BlockSpec((1, bsz), lambda i: (0, i)),
            pl.BlockSpec((1, bsz), lambda i: (0, i)),
            pl.BlockSpec((1, bsz), lambda i: (0, i)),
            pl.BlockSpec((1, bsz), lambda i: (0, i)),
            pl.BlockSpec((_STYLE_OFF, d_inst), lambda i: (0, 0)),
            pl.BlockSpec((d_inst, H), lambda i: (0, 0)),
            pl.BlockSpec((64, d_style), lambda i: (0, 0)),
            pl.BlockSpec((d_style, H), lambda i: (0, 0)),
            pl.BlockSpec((64, 192), lambda i: (0, 0)),
            pl.BlockSpec((192, H), lambda i: (0, 0)),
            pl.BlockSpec((64, 8), lambda i: (0, 0)),
            pl.BlockSpec((8, H), lambda i: (0, 0)),
        ],
        out_specs=pl.BlockSpec((bsz, H), lambda i: (i, 0)),
        out_shape=jax.ShapeDtypeStruct((B, H), f32),
        scratch_shapes=[pltpu.VMEM((_K, H), jnp.bfloat16)],
        compiler_params=pltpu.CompilerParams(
            dimension_semantics=("arbitrary",)),
    )(inst_t, sty_t, tempo_t, pitch_t, dur_t,
      inst_pad, wpi_t, style_pad, wps_t, v2, wptpd_t, e_one, b_proj2)
    return out
